# baseline (speedup 1.0000x reference)
import functools

import jax
import jax.numpy as jnp
from jax import lax
from jax.experimental import pallas as pl
from jax.experimental.pallas import tpu as pltpu
from jax.experimental.pallas import tpu_sc as plsc

VOCAB = 1000000
DIM = 64
BATCH = 4096
TOKENS = 200

_INFO = plsc.get_sparse_core_info()
_NC, _NS = _INFO.num_cores, _INFO.num_subcores
_NW = _NC * _NS
_TOTAL = BATCH * TOKENS
_GROUP = 128
_GROUPS_PER_W = _TOTAL // (_NW * _GROUP)


def _body(idx_hbm, table_hbm, out_hbm, idx_v, rows_v, gsem):
    wid = lax.axis_index("s") * _NC + lax.axis_index("c")
    row0 = wid * _GROUPS_PER_W
    pltpu.sync_copy(idx_hbm.at[pl.ds(row0, _GROUPS_PER_W)], idx_v)

    def step(j, carry):
        pltpu.async_copy(table_hbm.at[idx_v.at[j]], rows_v, gsem).wait()
        pltpu.sync_copy(rows_v, out_hbm.at[pl.ds((row0 + j) * _GROUP, _GROUP)])
        return carry

    lax.fori_loop(0, _GROUPS_PER_W, step, 0)


@functools.partial(
    pl.kernel,
    mesh=plsc.VectorSubcoreMesh(core_axis_name="c", subcore_axis_name="s"),
    out_type=jax.ShapeDtypeStruct((_TOTAL, DIM), jnp.float32),
    scratch_types=[
        pltpu.VMEM((_GROUPS_PER_W, _GROUP), jnp.int32),
        pltpu.VMEM((_GROUP, DIM), jnp.float32),
        pltpu.SemaphoreType.DMA,
    ],
    compiler_params=pltpu.CompilerParams(use_tc_tiling_on_sc=False),
)
def _gather_kernel(idx_hbm, table_hbm, out_hbm, idx_v, rows_v, gsem):
    _body(idx_hbm, table_hbm, out_hbm, idx_v, rows_v, gsem)


def kernel(inputs, embed_weight):
    idx = inputs.astype(jnp.int32).reshape(_TOTAL // _GROUP, _GROUP)
    out = _gather_kernel(idx, embed_weight)
    return out.reshape(BATCH, TOKENS, DIM)


# speedup vs baseline: 1.1149x; 1.1149x over previous
import functools

import jax
import jax.numpy as jnp
from jax import lax
from jax.experimental import pallas as pl
from jax.experimental.pallas import tpu as pltpu
from jax.experimental.pallas import tpu_sc as plsc

VOCAB = 1000000
DIM = 64
BATCH = 4096
TOKENS = 200

_INFO = plsc.get_sparse_core_info()
_NC, _NS = _INFO.num_cores, _INFO.num_subcores
_NW = _NC * _NS
_TOTAL = BATCH * TOKENS
_GROUP = 128
_GROUPS_PER_W = _TOTAL // (_NW * _GROUP)
_NBUF = 4
_BLOCKS = _GROUPS_PER_W // _NBUF


def _body(idx_hbm, table_hbm, out_hbm, idx_v, rows_v, gsems, ssems):
    wid = lax.axis_index("s") * _NC + lax.axis_index("c")
    row0 = wid * _GROUPS_PER_W
    pltpu.sync_copy(idx_hbm.at[pl.ds(row0, _GROUPS_PER_W)], idx_v)

    def buf(b):
        return rows_v.at[pl.ds(b * _GROUP, _GROUP)]

    def fire_gather(t, b):
        pltpu.async_copy(table_hbm.at[idx_v.at[t]], buf(b), gsems[b])

    def wait_gather(b):
        pltpu.make_async_copy(table_hbm.at[idx_v.at[0]], buf(b), gsems[b]).wait()

    def fire_store(t, b):
        pltpu.async_copy(buf(b), out_hbm.at[pl.ds((row0 + t) * _GROUP, _GROUP)],
                         ssems[b])

    def wait_store(b):
        pltpu.make_async_copy(
            buf(b), out_hbm.at[pl.ds(row0 * _GROUP, _GROUP)], ssems[b]).wait()

    for b in range(_NBUF):
        fire_gather(b, b)

    def block(gi, carry):
        g = gi * _NBUF
        for b in range(_NBUF):
            t = g + b
            wait_gather(b)
            fire_store(t, b)
            wait_store(b)
            fire_gather(t + _NBUF, b)
        return carry

    lax.fori_loop(0, _BLOCKS - 1, block, 0)
    g = (_BLOCKS - 1) * _NBUF
    for b in range(_NBUF):
        wait_gather(b)
        fire_store(g + b, b)
    for b in range(_NBUF):
        wait_store(b)


@functools.partial(
    pl.kernel,
    mesh=plsc.VectorSubcoreMesh(core_axis_name="c", subcore_axis_name="s"),
    out_type=jax.ShapeDtypeStruct((_TOTAL, DIM), jnp.float32),
    scratch_types=[
        pltpu.VMEM((_GROUPS_PER_W, _GROUP), jnp.int32),
        pltpu.VMEM((_NBUF * _GROUP, DIM), jnp.float32),
        [pltpu.SemaphoreType.DMA] * _NBUF,
        [pltpu.SemaphoreType.DMA] * _NBUF,
    ],
    compiler_params=pltpu.CompilerParams(use_tc_tiling_on_sc=False),
)
def _gather_kernel(idx_hbm, table_hbm, out_hbm, idx_v, rows_v, gsems, ssems):
    _body(idx_hbm, table_hbm, out_hbm, idx_v, rows_v, gsems, ssems)


def kernel(inputs, embed_weight):
    idx = inputs.astype(jnp.int32).reshape(_TOTAL // _GROUP, _GROUP)
    out = _gather_kernel(idx, embed_weight)
    return out.reshape(BATCH, TOKENS, DIM)


# speedup vs baseline: 1.3615x; 1.2212x over previous
import functools

import jax
import jax.numpy as jnp
from jax import lax
from jax.experimental import pallas as pl
from jax.experimental.pallas import tpu as pltpu
from jax.experimental.pallas import tpu_sc as plsc

VOCAB = 1000000
DIM = 64
PAD_DIM = 128
BATCH = 4096
TOKENS = 200

_INFO = plsc.get_sparse_core_info()
_NC, _NS = _INFO.num_cores, _INFO.num_subcores
_NW = _NC * _NS
_TOTAL = BATCH * TOKENS
_GROUP = 128
_GROUPS_PER_W = _TOTAL // (_NW * _GROUP)
_NBUF = 4
_BLOCKS = _GROUPS_PER_W // _NBUF


def _body(idx_hbm, table_hbm, out_hbm, idx_v, rows_v, gsems, ssems):
    wid = lax.axis_index("s") * _NC + lax.axis_index("c")
    row0 = wid * _GROUPS_PER_W
    pltpu.sync_copy(idx_hbm.at[pl.ds(row0, _GROUPS_PER_W)], idx_v)

    def buf(b):
        return rows_v.at[pl.ds(b * _GROUP, _GROUP)]

    def fire_gather(t, b):
        pltpu.async_copy(table_hbm.at[idx_v.at[t]], buf(b), gsems[b])

    def wait_gather(b):
        pltpu.make_async_copy(table_hbm.at[idx_v.at[0]], buf(b), gsems[b]).wait()

    def fire_store(t, b):
        pltpu.async_copy(buf(b), out_hbm.at[pl.ds((row0 + t) * _GROUP, _GROUP)],
                         ssems[b])

    def wait_store(b):
        pltpu.make_async_copy(
            buf(b), out_hbm.at[pl.ds(row0 * _GROUP, _GROUP)], ssems[b]).wait()

    for b in range(_NBUF):
        fire_gather(b, b)

    def block(gi, carry):
        g = gi * _NBUF
        for b in range(_NBUF):
            t = g + b
            wait_gather(b)
            fire_store(t, b)
            wait_store(b)
            fire_gather(t + _NBUF, b)
        return carry

    lax.fori_loop(0, _BLOCKS - 1, block, 0)
    g = (_BLOCKS - 1) * _NBUF
    for b in range(_NBUF):
        wait_gather(b)
        fire_store(g + b, b)
    for b in range(_NBUF):
        wait_store(b)


@functools.partial(
    pl.kernel,
    mesh=plsc.VectorSubcoreMesh(core_axis_name="c", subcore_axis_name="s"),
    out_type=jax.ShapeDtypeStruct((_TOTAL, PAD_DIM), jnp.float32),
    scratch_types=[
        pltpu.VMEM((_GROUPS_PER_W, _GROUP), jnp.int32),
        pltpu.VMEM((_NBUF * _GROUP, PAD_DIM), jnp.float32),
        [pltpu.SemaphoreType.DMA] * _NBUF,
        [pltpu.SemaphoreType.DMA] * _NBUF,
    ],
)
def _gather_kernel(idx_hbm, table_hbm, out_hbm, idx_v, rows_v, gsems, ssems):
    _body(idx_hbm, table_hbm, out_hbm, idx_v, rows_v, gsems, ssems)


def kernel(inputs, embed_weight):
    idx = inputs.astype(jnp.int32).reshape(_TOTAL // _GROUP, _GROUP)
    table = jnp.pad(embed_weight, ((0, 0), (0, PAD_DIM - DIM)))
    out = _gather_kernel(idx, table)
    return out[:, :DIM].reshape(BATCH, TOKENS, DIM)


# speedup vs baseline: 1.4714x; 1.0807x over previous
import functools

import jax
import jax.numpy as jnp
from jax import lax
from jax.experimental import pallas as pl
from jax.experimental.pallas import tpu as pltpu
from jax.experimental.pallas import tpu_sc as plsc

VOCAB = 1000000
DIM = 64
PAD_DIM = 128
BATCH = 4096
TOKENS = 200

_INFO = plsc.get_sparse_core_info()
_NC, _NS = _INFO.num_cores, _INFO.num_subcores
_NW = _NC * _NS
_TOTAL = BATCH * TOKENS
_GROUP = 128
_GROUPS_PER_W = _TOTAL // (_NW * _GROUP)
_NBUF = 4
_BLOCKS = _GROUPS_PER_W // _NBUF


def _body(idx_hbm, table_hbm, out_hbm, idx_v, rows_v, gsems, ssems):
    wid = lax.axis_index("s") * _NC + lax.axis_index("c")
    row0 = wid * _GROUPS_PER_W
    pltpu.sync_copy(idx_hbm.at[pl.ds(row0, _GROUPS_PER_W)], idx_v)

    def buf(b):
        return rows_v.at[pl.ds(b * _GROUP, _GROUP)]

    def buf_data(b):
        return rows_v.at[pl.ds(b * _GROUP, _GROUP), pl.ds(0, DIM)]

    def fire_gather(t, b):
        pltpu.async_copy(table_hbm.at[idx_v.at[t]], buf(b), gsems[b])

    def wait_gather(b):
        pltpu.make_async_copy(table_hbm.at[idx_v.at[0]], buf(b), gsems[b]).wait()

    def fire_store(t, b):
        pltpu.async_copy(buf_data(b),
                         out_hbm.at[pl.ds((row0 + t) * _GROUP, _GROUP),
                                    pl.ds(0, DIM)],
                         ssems[b])

    def wait_store(b):
        pltpu.make_async_copy(
            buf_data(b),
            out_hbm.at[pl.ds(row0 * _GROUP, _GROUP), pl.ds(0, DIM)],
            ssems[b]).wait()

    for b in range(_NBUF):
        fire_gather(b, b)

    def block(gi, carry):
        g = gi * _NBUF
        for b in range(_NBUF):
            t = g + b
            wait_gather(b)
            fire_store(t, b)
            wait_store(b)
            fire_gather(t + _NBUF, b)
        return carry

    lax.fori_loop(0, _BLOCKS - 1, block, 0)
    g = (_BLOCKS - 1) * _NBUF
    for b in range(_NBUF):
        wait_gather(b)
        fire_store(g + b, b)
    for b in range(_NBUF):
        wait_store(b)


@functools.partial(
    pl.kernel,
    mesh=plsc.VectorSubcoreMesh(core_axis_name="c", subcore_axis_name="s"),
    out_type=jax.ShapeDtypeStruct((_TOTAL, PAD_DIM), jnp.float32),
    scratch_types=[
        pltpu.VMEM((_GROUPS_PER_W, _GROUP), jnp.int32),
        pltpu.VMEM((_NBUF * _GROUP, PAD_DIM), jnp.float32),
        [pltpu.SemaphoreType.DMA] * _NBUF,
        [pltpu.SemaphoreType.DMA] * _NBUF,
    ],
    compiler_params=pltpu.CompilerParams(use_tc_tiling_on_sc=False),
)
def _gather_kernel(idx_hbm, table_hbm, out_hbm, idx_v, rows_v, gsems, ssems):
    _body(idx_hbm, table_hbm, out_hbm, idx_v, rows_v, gsems, ssems)


def kernel(inputs, embed_weight):
    idx = inputs.astype(jnp.int32).reshape(_TOTAL // _GROUP, _GROUP)
    table = jnp.pad(embed_weight, ((0, 0), (0, PAD_DIM - DIM)))
    out = _gather_kernel(idx, table)
    return out[:, :DIM].reshape(BATCH, TOKENS, DIM)
